# baseline (speedup 1.0000x reference)
import functools

import jax
import jax.numpy as jnp
from jax import lax
from jax.experimental import pallas as pl
from jax.experimental.pallas import tpu as pltpu
from jax.experimental.pallas import tpu_sc as plsc

SR6 = 1.281
SR8 = 1.094
NELEM = 95

NC = 2
NS = 16
NW = NC * NS
L = 16

TABLE_PAD = 9040
CHUNK = 6400
UNROLL = 16


def _sc_zero_damp(P, species12, distances, table, coef_a, flag):
    n_chunks = P // CHUNK
    n_pairs = -(-n_chunks // (2 * NW))

    mesh = plsc.VectorSubcoreMesh(core_axis_name="c", subcore_axis_name="s")

    @functools.partial(
        pl.kernel,
        mesh=mesh,
        compiler_params=pltpu.CompilerParams(needs_layout_passes=False),
        out_type=jax.ShapeDtypeStruct((P,), jnp.float32),
        scratch_types=[
            pltpu.VMEM((TABLE_PAD,), jnp.float32),
            pltpu.VMEM((L,), jnp.float32),
            pltpu.VMEM((L,), jnp.float32),
            pltpu.VMEM((2, CHUNK), jnp.int32),
            pltpu.VMEM((2, CHUNK), jnp.int32),
            pltpu.VMEM((CHUNK,), jnp.float32),
            pltpu.VMEM((CHUNK,), jnp.float32),
            pltpu.VMEM((CHUNK,), jnp.float32),
            pltpu.VMEM((CHUNK,), jnp.float32),
            pltpu.SemaphoreType.DMA,
            pltpu.SemaphoreType.DMA,
            pltpu.SemaphoreType.DMA,
            pltpu.SemaphoreType.DMA,
            pltpu.SemaphoreType.DMA,
        ],
    )
    def k(s12_hbm, d_hbm, table_hbm, coef_hbm, flag_hbm, out_hbm,
          table_v, coef_v, flag_v, s12a, s12b, da, db, oa, ob,
          sia, sib, soa, sob, sp):
        wid = lax.axis_index("c") * NS + lax.axis_index("s")

        def in_copies(c, s12_v, d_v, sem):
            off = c * CHUNK
            return (
                pltpu.make_async_copy(
                    s12_hbm.at[:, pl.ds(off, CHUNK)], s12_v, sem),
                pltpu.make_async_copy(d_hbm.at[pl.ds(off, CHUNK)], d_v, sem),
            )

        def out_copy(c, o_v, sem):
            return pltpu.make_async_copy(
                o_v, out_hbm.at[pl.ds(c * CHUNK, CHUNK)], sem)

        def compute(s12_v, d_v, o_v):
            @plsc.parallel_loop(0, CHUNK, step=L, unroll=UNROLL)
            def do_vec(i):
                s0 = s12_v[0, pl.ds(i, L)]
                s1 = s12_v[1, pl.ds(i, L)]
                d = d_v[pl.ds(i, L)]
                idx = s0 * NELEM + s1
                cr = plsc.load_gather(table_v, [idx])
                m = cr / (cr + coef * d)
                w = d * m
                w2 = w * w
                m2 = m * m
                m4 = m2 * m2
                w4 = w2 * w2
                r6 = (w4 * w2) * (m4 * m4)
                o_v[pl.ds(i, L)] = jnp.where(use_r8, r6 * w2, r6)

        def process(kk, c, s12_v, d_v, o_v, si, so):
            for cp in in_copies(c, s12_v, d_v, si):
                cp.wait()

            @pl.when(kk > 0)
            def _():
                out_copy(c - 2 * NW, o_v, so).wait()

            compute(s12_v, d_v, o_v)
            out_copy(c, o_v, so).start()

        prologue = [
            pltpu.make_async_copy(table_hbm, table_v, sp),
            pltpu.make_async_copy(coef_hbm, coef_v, sp),
            pltpu.make_async_copy(flag_hbm, flag_v, sp),
        ]
        for cp in prologue:
            cp.start()
        for cp in in_copies(wid, s12a, da, sia):
            cp.start()
        for cp in prologue:
            cp.wait()

        coef = coef_v[...]
        use_r8 = flag_v[...] < 0.5

        def body(kk, carry):
            c0 = wid + (2 * kk) * NW
            c1 = c0 + NW
            c0n = c0 + 2 * NW

            @pl.when(c1 < n_chunks)
            def _():
                for cp in in_copies(c1, s12b, db, sib):
                    cp.start()

            process(kk, c0, s12a, da, oa, sia, soa)

            @pl.when(c0n < n_chunks)
            def _():
                for cp in in_copies(c0n, s12a, da, sia):
                    cp.start()

            @pl.when(c1 < n_chunks)
            def _():
                process(kk, c1, s12b, db, ob, sib, sob)

            return carry

        lax.fori_loop(0, n_pairs, body, 0)
        out_copy(0, oa, soa).wait()
        out_copy(0, ob, sob).wait()

    return k(species12, distances, table, coef_a, flag)


def kernel(species12, distances, cutoff_radii, order):
    P = distances.shape[0]
    order6 = order == 6
    sr = jnp.where(order6, SR6, SR8).astype(jnp.float32)
    coef_a = jnp.full((L,), 6.0, jnp.float32) / sr
    flag = jnp.where(order6, 1.0, 0.0).astype(jnp.float32) * jnp.ones(
        (L,), jnp.float32)
    table = jnp.pad(cutoff_radii.reshape(-1).astype(jnp.float32),
                    (0, TABLE_PAD - NELEM * NELEM))
    return _sc_zero_damp(P, species12, distances, table, coef_a, flag)


# speedup vs baseline: 1.1153x; 1.1153x over previous
import functools

import jax
import jax.numpy as jnp
from jax import lax
from jax.experimental import pallas as pl
from jax.experimental.pallas import tpu as pltpu
from jax.experimental.pallas import tpu_sc as plsc

SR6 = 1.281
SR8 = 1.094
NELEM = 95

NC = 2
NS = 16
NW = NC * NS
L = 16

TABLE_PAD = 9040
CHUNK = 6400
UNROLL = 4


def _sc_zero_damp(P, species12, distances, table, coef_a, flag):
    n_chunks = P // CHUNK
    n_pairs = -(-n_chunks // (2 * NW))

    mesh = plsc.VectorSubcoreMesh(core_axis_name="c", subcore_axis_name="s")

    @functools.partial(
        pl.kernel,
        mesh=mesh,
        compiler_params=pltpu.CompilerParams(needs_layout_passes=False),
        out_type=jax.ShapeDtypeStruct((P,), jnp.float32),
        scratch_types=[
            pltpu.VMEM((TABLE_PAD,), jnp.float32),
            pltpu.VMEM((L,), jnp.float32),
            pltpu.VMEM((L,), jnp.float32),
            pltpu.VMEM((2, CHUNK), jnp.int32),
            pltpu.VMEM((2, CHUNK), jnp.int32),
            pltpu.VMEM((CHUNK,), jnp.float32),
            pltpu.VMEM((CHUNK,), jnp.float32),
            pltpu.VMEM((CHUNK,), jnp.float32),
            pltpu.VMEM((CHUNK,), jnp.float32),
            pltpu.SemaphoreType.DMA,
            pltpu.SemaphoreType.DMA,
            pltpu.SemaphoreType.DMA,
            pltpu.SemaphoreType.DMA,
            pltpu.SemaphoreType.DMA,
        ],
    )
    def k(s12_hbm, d_hbm, table_hbm, coef_hbm, flag_hbm, out_hbm,
          table_v, coef_v, flag_v, s12a, s12b, da, db, oa, ob,
          sia, sib, soa, sob, sp):
        wid = lax.axis_index("c") * NS + lax.axis_index("s")

        def in_copies(c, s12_v, d_v, sem):
            off = c * CHUNK
            return (
                pltpu.make_async_copy(
                    s12_hbm.at[:, pl.ds(off, CHUNK)], s12_v, sem),
                pltpu.make_async_copy(d_hbm.at[pl.ds(off, CHUNK)], d_v, sem),
            )

        def out_copy(c, o_v, sem):
            return pltpu.make_async_copy(
                o_v, out_hbm.at[pl.ds(c * CHUNK, CHUNK)], sem)

        def compute(s12_v, d_v, o_v):
            @plsc.parallel_loop(0, CHUNK, step=L, unroll=UNROLL)
            def do_vec(i):
                s0 = s12_v[0, pl.ds(i, L)]
                s1 = s12_v[1, pl.ds(i, L)]
                d = d_v[pl.ds(i, L)]
                idx = s0 * NELEM + s1
                cr = plsc.load_gather(table_v, [idx])
                m = cr / (cr + coef * d)
                w = d * m
                w2 = w * w
                m2 = m * m
                m4 = m2 * m2
                w4 = w2 * w2
                r6 = (w4 * w2) * (m4 * m4)
                o_v[pl.ds(i, L)] = jnp.where(use_r8, r6 * w2, r6)

        def process(kk, c, s12_v, d_v, o_v, si, so):
            for cp in in_copies(c, s12_v, d_v, si):
                cp.wait()

            @pl.when(kk > 0)
            def _():
                out_copy(c - 2 * NW, o_v, so).wait()

            compute(s12_v, d_v, o_v)
            out_copy(c, o_v, so).start()

        prologue = [
            pltpu.make_async_copy(table_hbm, table_v, sp),
            pltpu.make_async_copy(coef_hbm, coef_v, sp),
            pltpu.make_async_copy(flag_hbm, flag_v, sp),
        ]
        for cp in prologue:
            cp.start()
        for cp in in_copies(wid, s12a, da, sia):
            cp.start()
        for cp in prologue:
            cp.wait()

        coef = coef_v[...]
        use_r8 = flag_v[...] < 0.5

        def body(kk, carry):
            c0 = wid + (2 * kk) * NW
            c1 = c0 + NW
            c0n = c0 + 2 * NW

            @pl.when(c1 < n_chunks)
            def _():
                for cp in in_copies(c1, s12b, db, sib):
                    cp.start()

            process(kk, c0, s12a, da, oa, sia, soa)

            @pl.when(c0n < n_chunks)
            def _():
                for cp in in_copies(c0n, s12a, da, sia):
                    cp.start()

            @pl.when(c1 < n_chunks)
            def _():
                process(kk, c1, s12b, db, ob, sib, sob)

            return carry

        lax.fori_loop(0, n_pairs, body, 0)
        out_copy(0, oa, soa).wait()
        out_copy(0, ob, sob).wait()

    return k(species12, distances, table, coef_a, flag)


def kernel(species12, distances, cutoff_radii, order):
    P = distances.shape[0]
    order6 = order == 6
    sr = jnp.where(order6, SR6, SR8).astype(jnp.float32)
    coef_a = jnp.full((L,), 6.0, jnp.float32) / sr
    flag = jnp.where(order6, 1.0, 0.0).astype(jnp.float32) * jnp.ones(
        (L,), jnp.float32)
    table = jnp.pad(cutoff_radii.reshape(-1).astype(jnp.float32),
                    (0, TABLE_PAD - NELEM * NELEM))
    return _sc_zero_damp(P, species12, distances, table, coef_a, flag)
